# baseline (speedup 1.0000x reference)
import functools

import jax
import jax.numpy as jnp
from jax import lax
from jax.experimental import pallas as pl
from jax.experimental.pallas import tpu as pltpu
from jax.experimental.pallas import tpu_sc as plsc

N = 10000
E = 320000
D = 128

NC = 2
NS = 16
NW = NC * NS

EPW = E // NW
C = 64
CH = EPW // C
TE = EPW - CH * C
RPT = 624
TAIL = N - NS * RPT


def _mm_body(x_ref, w_ref, o_ref):
    o_ref[...] = jnp.dot(x_ref[...], w_ref[...],
                         preferred_element_type=jnp.float32)


def _matmul(x, w):
    return pl.pallas_call(
        _mm_body,
        grid=(10,),
        in_specs=[
            pl.BlockSpec((N // 10, D), lambda i: (i, 0)),
            pl.BlockSpec((D, D), lambda i: (0, 0)),
        ],
        out_specs=pl.BlockSpec((N // 10, D), lambda i: (i, 0)),
        out_shape=jax.ShapeDtypeStruct((N, D), jnp.float32),
    )(x, w)


def _comb_body(p_ref, o_ref):
    o_ref[...] = jnp.maximum(p_ref[0] + p_ref[1], 0.0)


def _combine(p):
    return pl.pallas_call(
        _comb_body,
        grid=(10,),
        in_specs=[pl.BlockSpec((2, N // 10, D), lambda i: (0, i, 0))],
        out_specs=pl.BlockSpec((N // 10, D), lambda i: (i, 0)),
        out_shape=jax.ShapeDtypeStruct((N, D), jnp.float32),
    )(p)


def _bcast_lane(vec, k):
    return lax.gather(
        vec, jnp.full((16, 1), k, jnp.int32),
        lax.GatherDimensionNumbers(
            offset_dims=(), collapsed_slice_dims=(0,), start_index_map=(0,)),
        (1,), mode=lax.GatherScatterMode.PROMISE_IN_BOUNDS)


def _scale_rows(rows_ref, ev_ref, ngroups):
    def _group(g, carry):
        ev_vec = ev_ref[pl.ds(g * 16, 16)]
        for k in range(16):
            s = _bcast_lane(ev_vec, k)
            e = g * 16 + k
            for j in range(D // 16):
                rows_ref[e, pl.ds(j * 16, 16)] = (
                    rows_ref[e, pl.ds(j * 16, 16)] * s)
        return carry

    lax.fori_loop(0, ngroups, _group, 0)


@functools.partial(
    pl.kernel,
    out_type=jax.ShapeDtypeStruct((NC, N, D), jnp.float32),
    mesh=plsc.VectorSubcoreMesh(core_axis_name="c", subcore_axis_name="s"),
    scratch_types=(
        [pltpu.VMEM((C,), jnp.int32) for _ in range(3)]
        + [pltpu.VMEM((C,), jnp.int32) for _ in range(3)]
        + [pltpu.VMEM((C,), jnp.float32) for _ in range(3)]
        + [pltpu.VMEM((C, D), jnp.float32) for _ in range(2)]
        + [
            pltpu.VMEM((TE,), jnp.int32),
            pltpu.VMEM((TE,), jnp.int32),
            pltpu.VMEM((TE,), jnp.float32),
            pltpu.VMEM((TE, D), jnp.float32),
            pltpu.VMEM_SHARED((N, D), jnp.float32),
        ]
        + [pltpu.SemaphoreType.DMA for _ in range(7)]
    ),
)
def _sc_aggregate(src_hbm, dst_hbm, ev_hbm, xw_hbm, out_hbm, *sc):
    srcs, dsts, evs = sc[0:3], sc[3:6], sc[6:9]
    rows = sc[9:11]
    src_t, dst_t, ev_t, rows_t, acc = sc[11:16]
    gsem, isem, ssem = sc[16:18], sc[18:21], sc[21:23]

    cid = lax.axis_index("c")
    sid = lax.axis_index("s")
    wid = cid * NS + sid
    ebase = wid * EPW
    row0 = sid * RPT

    def _idx_issue(k, slot):
        base = ebase + k * C
        pltpu.async_copy(src_hbm.at[pl.ds(base, C)], srcs[slot], isem[slot])
        pltpu.async_copy(dst_hbm.at[pl.ds(base, C)], dsts[slot], isem[slot])
        pltpu.async_copy(ev_hbm.at[pl.ds(base, C)], evs[slot], isem[slot])

    def _idx_wait(slot):
        pltpu.make_async_copy(src_hbm.at[pl.ds(0, C)], srcs[slot], isem[slot]).wait()
        pltpu.make_async_copy(dst_hbm.at[pl.ds(0, C)], dsts[slot], isem[slot]).wait()
        pltpu.make_async_copy(ev_hbm.at[pl.ds(0, C)], evs[slot], isem[slot]).wait()

    _idx_issue(0, 0)
    _idx_issue(1, 1)

    def _zstore(r, carry):
        for j in range(D // 16):
            rows[0][r, pl.ds(j * 16, 16)] = jnp.zeros((16,), jnp.float32)
        return carry

    lax.fori_loop(0, C, _zstore, 0)
    for zb in list(range(0, RPT - C + 1, C)) + ([RPT - C] if RPT % C else []):
        pltpu.sync_copy(rows[0], acc.at[pl.ds(row0 + zb, C)])

    @pl.when(sid == NS - 1)
    def _ztail():
        pltpu.sync_copy(rows[0].at[pl.ds(0, TAIL)],
                        acc.at[pl.ds(NS * RPT, TAIL)])

    _idx_wait(0)
    pltpu.async_copy(xw_hbm.at[srcs[0]], rows[0], gsem[0])
    plsc.subcore_barrier()

    def _step(i, u):
        b = u % 2
        nb = (u + 1) % 2
        s_i = u % 3
        s_n1 = (u + 1) % 3
        s_n2 = (u + 2) % 3

        @pl.when(i >= 1)
        def _():
            pltpu.make_async_copy(rows[nb], acc.at[dsts[s_n2]], ssem[nb]).wait()

        @pl.when(i + 1 < CH)
        def _():
            _idx_wait(s_n1)
            pltpu.async_copy(xw_hbm.at[srcs[s_n1]], rows[nb], gsem[nb])

        @pl.when(i + 2 < CH)
        def _():
            _idx_issue(i + 2, s_n2)

        pltpu.make_async_copy(xw_hbm.at[srcs[s_i]], rows[b], gsem[b]).wait()
        _scale_rows(rows[b], evs[s_i], C // 16)
        pltpu.async_copy(rows[b], acc.at[dsts[s_i]], ssem[b], add=True)

    def _outer(t, carry):
        for u in range(6):
            _step(6 * t + u, u)
        return carry

    lax.fori_loop(0, CH // 6, _outer, 0)

    pltpu.make_async_copy(rows[(CH - 1) % 2], acc.at[dsts[(CH - 1) % 3]],
                          ssem[(CH - 1) % 2]).wait()

    tbase = ebase + CH * C
    pltpu.sync_copy(src_hbm.at[pl.ds(tbase, TE)], src_t)
    pltpu.sync_copy(dst_hbm.at[pl.ds(tbase, TE)], dst_t)
    pltpu.sync_copy(ev_hbm.at[pl.ds(tbase, TE)], ev_t)
    pltpu.async_copy(xw_hbm.at[src_t], rows_t, gsem[0]).wait()
    _scale_rows(rows_t, ev_t, TE // 16)
    pltpu.sync_copy(rows_t, acc.at[dst_t], add=True)

    plsc.subcore_barrier()

    pltpu.sync_copy(acc.at[pl.ds(row0, RPT)], out_hbm.at[cid, pl.ds(row0, RPT)])

    @pl.when(sid == NS - 1)
    def _wtail():
        pltpu.sync_copy(acc.at[pl.ds(NS * RPT, TAIL)],
                        out_hbm.at[cid, pl.ds(NS * RPT, TAIL)])


def kernel(x, edge_index, edge_values, W):
    xw = _matmul(x, W)
    dst = edge_index[0]
    src = edge_index[1]
    partials = _sc_aggregate(src, dst, edge_values, xw)
    return _combine(partials)


# speedup vs baseline: 1.1136x; 1.1136x over previous
import functools

import jax
import jax.numpy as jnp
from jax import lax
from jax.experimental import pallas as pl
from jax.experimental.pallas import tpu as pltpu
from jax.experimental.pallas import tpu_sc as plsc

N = 10000
E = 320000
D = 128

NC = 2
NS = 16
NW = NC * NS

EPW = E // NW
C = 128
CH = EPW // C
TE = EPW - CH * C
RPT = 624
TAIL = N - NS * RPT


def _mm_body(x_ref, w_ref, o_ref):
    o_ref[...] = jnp.dot(x_ref[...], w_ref[...],
                         preferred_element_type=jnp.float32)


def _matmul(x, w):
    return pl.pallas_call(
        _mm_body,
        grid=(10,),
        in_specs=[
            pl.BlockSpec((N // 10, D), lambda i: (i, 0)),
            pl.BlockSpec((D, D), lambda i: (0, 0)),
        ],
        out_specs=pl.BlockSpec((N // 10, D), lambda i: (i, 0)),
        out_shape=jax.ShapeDtypeStruct((N, D), jnp.float32),
    )(x, w)


def _comb_body(p_ref, o_ref):
    o_ref[...] = jnp.maximum(p_ref[0] + p_ref[1], 0.0)


def _combine(p):
    return pl.pallas_call(
        _comb_body,
        grid=(10,),
        in_specs=[pl.BlockSpec((2, N // 10, D), lambda i: (0, i, 0))],
        out_specs=pl.BlockSpec((N // 10, D), lambda i: (i, 0)),
        out_shape=jax.ShapeDtypeStruct((N, D), jnp.float32),
    )(p)


def _bcast_lane(vec, k):
    return lax.gather(
        vec, jnp.full((16, 1), k, jnp.int32),
        lax.GatherDimensionNumbers(
            offset_dims=(), collapsed_slice_dims=(0,), start_index_map=(0,)),
        (1,), mode=lax.GatherScatterMode.PROMISE_IN_BOUNDS)


def _scale_rows(rows_ref, ev_ref, ngroups):
    def _group(g, carry):
        ev_vec = ev_ref[pl.ds(g * 16, 16)]
        for k in range(16):
            s = _bcast_lane(ev_vec, k)
            e = g * 16 + k
            for j in range(D // 16):
                rows_ref[e, pl.ds(j * 16, 16)] = (
                    rows_ref[e, pl.ds(j * 16, 16)] * s)
        return carry

    lax.fori_loop(0, ngroups, _group, 0)


@functools.partial(
    pl.kernel,
    out_type=jax.ShapeDtypeStruct((NC, N, D), jnp.float32),
    mesh=plsc.VectorSubcoreMesh(core_axis_name="c", subcore_axis_name="s"),
    scratch_types=(
        [pltpu.VMEM((C,), jnp.int32) for _ in range(3)]
        + [pltpu.VMEM((C,), jnp.int32) for _ in range(3)]
        + [pltpu.VMEM((C,), jnp.float32) for _ in range(3)]
        + [pltpu.VMEM((C, D), jnp.float32) for _ in range(2)]
        + [
            pltpu.VMEM((TE,), jnp.int32),
            pltpu.VMEM((TE,), jnp.int32),
            pltpu.VMEM((TE,), jnp.float32),
            pltpu.VMEM((TE, D), jnp.float32),
            pltpu.VMEM_SHARED((N, D), jnp.float32),
        ]
        + [pltpu.SemaphoreType.DMA for _ in range(7)]
    ),
)
def _sc_aggregate(src_hbm, dst_hbm, ev_hbm, xw_hbm, out_hbm, *sc):
    srcs, dsts, evs = sc[0:3], sc[3:6], sc[6:9]
    rows = sc[9:11]
    src_t, dst_t, ev_t, rows_t, acc = sc[11:16]
    gsem, isem, ssem = sc[16:18], sc[18:21], sc[21:23]

    cid = lax.axis_index("c")
    sid = lax.axis_index("s")
    wid = cid * NS + sid
    ebase = wid * EPW
    row0 = sid * RPT

    def _idx_issue(k, slot):
        base = ebase + k * C
        pltpu.async_copy(src_hbm.at[pl.ds(base, C)], srcs[slot], isem[slot])
        pltpu.async_copy(dst_hbm.at[pl.ds(base, C)], dsts[slot], isem[slot])
        pltpu.async_copy(ev_hbm.at[pl.ds(base, C)], evs[slot], isem[slot])

    def _idx_wait(slot):
        pltpu.make_async_copy(src_hbm.at[pl.ds(0, C)], srcs[slot], isem[slot]).wait()
        pltpu.make_async_copy(dst_hbm.at[pl.ds(0, C)], dsts[slot], isem[slot]).wait()
        pltpu.make_async_copy(ev_hbm.at[pl.ds(0, C)], evs[slot], isem[slot]).wait()

    _idx_issue(0, 0)
    _idx_issue(1, 1)

    def _zstore(r, carry):
        for j in range(D // 16):
            rows[0][r, pl.ds(j * 16, 16)] = jnp.zeros((16,), jnp.float32)
        return carry

    lax.fori_loop(0, C, _zstore, 0)
    for zb in list(range(0, RPT - C + 1, C)) + ([RPT - C] if RPT % C else []):
        pltpu.sync_copy(rows[0], acc.at[pl.ds(row0 + zb, C)])

    @pl.when(sid == NS - 1)
    def _ztail():
        pltpu.sync_copy(rows[0].at[pl.ds(0, TAIL)],
                        acc.at[pl.ds(NS * RPT, TAIL)])

    _idx_wait(0)
    pltpu.async_copy(xw_hbm.at[srcs[0]], rows[0], gsem[0])
    plsc.subcore_barrier()

    def _step(i, u):
        b = u % 2
        nb = (u + 1) % 2
        s_i = u % 3
        s_n1 = (u + 1) % 3
        s_n2 = (u + 2) % 3

        @pl.when(i >= 1)
        def _():
            pltpu.make_async_copy(rows[nb], acc.at[dsts[s_n2]], ssem[nb]).wait()

        @pl.when(i + 1 < CH)
        def _():
            _idx_wait(s_n1)
            pltpu.async_copy(xw_hbm.at[srcs[s_n1]], rows[nb], gsem[nb])

        @pl.when(i + 2 < CH)
        def _():
            _idx_issue(i + 2, s_n2)

        pltpu.make_async_copy(xw_hbm.at[srcs[s_i]], rows[b], gsem[b]).wait()
        _scale_rows(rows[b], evs[s_i], C // 16)
        pltpu.async_copy(rows[b], acc.at[dsts[s_i]], ssem[b], add=True)

    def _outer(t, carry):
        for u in range(6):
            _step(6 * t + u, u)
        return carry

    lax.fori_loop(0, CH // 6, _outer, 0)

    pltpu.make_async_copy(rows[(CH - 1) % 2], acc.at[dsts[(CH - 1) % 3]],
                          ssem[(CH - 1) % 2]).wait()

    tbase = ebase + CH * C
    pltpu.sync_copy(src_hbm.at[pl.ds(tbase, TE)], src_t)
    pltpu.sync_copy(dst_hbm.at[pl.ds(tbase, TE)], dst_t)
    pltpu.sync_copy(ev_hbm.at[pl.ds(tbase, TE)], ev_t)
    pltpu.async_copy(xw_hbm.at[src_t], rows_t, gsem[0]).wait()
    _scale_rows(rows_t, ev_t, TE // 16)
    pltpu.sync_copy(rows_t, acc.at[dst_t], add=True)

    plsc.subcore_barrier()

    pltpu.sync_copy(acc.at[pl.ds(row0, RPT)], out_hbm.at[cid, pl.ds(row0, RPT)])

    @pl.when(sid == NS - 1)
    def _wtail():
        pltpu.sync_copy(acc.at[pl.ds(NS * RPT, TAIL)],
                        out_hbm.at[cid, pl.ds(NS * RPT, TAIL)])


def kernel(x, edge_index, edge_values, W):
    xw = _matmul(x, W)
    dst = edge_index[0]
    src = edge_index[1]
    partials = _sc_aggregate(src, dst, edge_values, xw)
    return _combine(partials)


# speedup vs baseline: 1.1643x; 1.0456x over previous
import functools

import jax
import jax.numpy as jnp
from jax import lax
from jax.experimental import pallas as pl
from jax.experimental.pallas import tpu as pltpu
from jax.experimental.pallas import tpu_sc as plsc

N = 10000
E = 320000
D = 128

NC = 2
NS = 16
NW = NC * NS

EPW = E // NW
C = 128
CH = EPW // C
TE = EPW - CH * C
RPT = 624
TAIL = N - NS * RPT


def _mm_body(x_ref, w_ref, o_ref):
    o_ref[...] = jnp.dot(x_ref[...], w_ref[...],
                         preferred_element_type=jnp.float32)


def _matmul(x, w):
    return pl.pallas_call(
        _mm_body,
        grid=(2,),
        in_specs=[
            pl.BlockSpec((N // 2, D), lambda i: (i, 0)),
            pl.BlockSpec((D, D), lambda i: (0, 0)),
        ],
        out_specs=pl.BlockSpec((N // 2, D), lambda i: (i, 0)),
        out_shape=jax.ShapeDtypeStruct((N, D), jnp.float32),
    )(x, w)


def _comb_body(p_ref, o_ref):
    o_ref[...] = jnp.maximum(p_ref[0] + p_ref[1], 0.0)


def _combine(p):
    return pl.pallas_call(
        _comb_body,
        grid=(2,),
        in_specs=[pl.BlockSpec((2, N // 2, D), lambda i: (0, i, 0))],
        out_specs=pl.BlockSpec((N // 2, D), lambda i: (i, 0)),
        out_shape=jax.ShapeDtypeStruct((N, D), jnp.float32),
    )(p)


def _bcast_lane(vec, k):
    return lax.gather(
        vec, jnp.full((16, 1), k, jnp.int32),
        lax.GatherDimensionNumbers(
            offset_dims=(), collapsed_slice_dims=(0,), start_index_map=(0,)),
        (1,), mode=lax.GatherScatterMode.PROMISE_IN_BOUNDS)


def _scale_rows(rows_ref, ev_ref, ngroups):
    def _group(g, carry):
        ev_vec = ev_ref[pl.ds(g * 16, 16)]
        for k in range(16):
            s = _bcast_lane(ev_vec, k)
            e = g * 16 + k
            for j in range(D // 16):
                rows_ref[e, pl.ds(j * 16, 16)] = (
                    rows_ref[e, pl.ds(j * 16, 16)] * s)
        return carry

    lax.fori_loop(0, ngroups, _group, 0)


@functools.partial(
    pl.kernel,
    out_type=jax.ShapeDtypeStruct((NC, N, D), jnp.float32),
    mesh=plsc.VectorSubcoreMesh(core_axis_name="c", subcore_axis_name="s"),
    scratch_types=(
        [pltpu.VMEM((C,), jnp.int32) for _ in range(3)]
        + [pltpu.VMEM((C,), jnp.int32) for _ in range(3)]
        + [pltpu.VMEM((C,), jnp.float32) for _ in range(3)]
        + [pltpu.VMEM((C, D), jnp.float32) for _ in range(2)]
        + [
            pltpu.VMEM((TE,), jnp.int32),
            pltpu.VMEM((TE,), jnp.int32),
            pltpu.VMEM((TE,), jnp.float32),
            pltpu.VMEM((TE, D), jnp.float32),
            pltpu.VMEM_SHARED((N, D), jnp.float32),
        ]
        + [pltpu.SemaphoreType.DMA for _ in range(7)]
    ),
)
def _sc_aggregate(src_hbm, dst_hbm, ev_hbm, xw_hbm, out_hbm, *sc):
    srcs, dsts, evs = sc[0:3], sc[3:6], sc[6:9]
    rows = sc[9:11]
    src_t, dst_t, ev_t, rows_t, acc = sc[11:16]
    gsem, isem, ssem = sc[16:18], sc[18:21], sc[21:23]

    cid = lax.axis_index("c")
    sid = lax.axis_index("s")
    wid = cid * NS + sid
    ebase = wid * EPW
    row0 = sid * RPT

    def _idx_issue(k, slot):
        base = ebase + k * C
        pltpu.async_copy(src_hbm.at[pl.ds(base, C)], srcs[slot], isem[slot])
        pltpu.async_copy(dst_hbm.at[pl.ds(base, C)], dsts[slot], isem[slot])
        pltpu.async_copy(ev_hbm.at[pl.ds(base, C)], evs[slot], isem[slot])

    def _idx_wait(slot):
        pltpu.make_async_copy(src_hbm.at[pl.ds(0, C)], srcs[slot], isem[slot]).wait()
        pltpu.make_async_copy(dst_hbm.at[pl.ds(0, C)], dsts[slot], isem[slot]).wait()
        pltpu.make_async_copy(ev_hbm.at[pl.ds(0, C)], evs[slot], isem[slot]).wait()

    _idx_issue(0, 0)
    _idx_issue(1, 1)

    def _zstore(r, carry):
        for j in range(D // 16):
            rows[0][r, pl.ds(j * 16, 16)] = jnp.zeros((16,), jnp.float32)
        return carry

    lax.fori_loop(0, C, _zstore, 0)
    for zb in list(range(0, RPT - C + 1, C)) + ([RPT - C] if RPT % C else []):
        pltpu.sync_copy(rows[0], acc.at[pl.ds(row0 + zb, C)])

    @pl.when(sid == NS - 1)
    def _ztail():
        pltpu.sync_copy(rows[0].at[pl.ds(0, TAIL)],
                        acc.at[pl.ds(NS * RPT, TAIL)])

    _idx_wait(0)
    pltpu.async_copy(xw_hbm.at[srcs[0]], rows[0], gsem[0])
    plsc.subcore_barrier()

    def _step(i, u):
        b = u % 2
        nb = (u + 1) % 2
        s_i = u % 3
        s_n1 = (u + 1) % 3
        s_n2 = (u + 2) % 3

        @pl.when(i >= 1)
        def _():
            pltpu.make_async_copy(rows[nb], acc.at[dsts[s_n2]], ssem[nb]).wait()

        @pl.when(i + 1 < CH)
        def _():
            _idx_wait(s_n1)
            pltpu.async_copy(xw_hbm.at[srcs[s_n1]], rows[nb], gsem[nb])

        @pl.when(i + 2 < CH)
        def _():
            _idx_issue(i + 2, s_n2)

        pltpu.make_async_copy(xw_hbm.at[srcs[s_i]], rows[b], gsem[b]).wait()
        _scale_rows(rows[b], evs[s_i], C // 16)
        pltpu.async_copy(rows[b], acc.at[dsts[s_i]], ssem[b], add=True)

    def _outer(t, carry):
        for u in range(6):
            _step(6 * t + u, u)
        return carry

    lax.fori_loop(0, CH // 6, _outer, 0)

    pltpu.make_async_copy(rows[(CH - 1) % 2], acc.at[dsts[(CH - 1) % 3]],
                          ssem[(CH - 1) % 2]).wait()

    tbase = ebase + CH * C
    pltpu.sync_copy(src_hbm.at[pl.ds(tbase, TE)], src_t)
    pltpu.sync_copy(dst_hbm.at[pl.ds(tbase, TE)], dst_t)
    pltpu.sync_copy(ev_hbm.at[pl.ds(tbase, TE)], ev_t)
    pltpu.async_copy(xw_hbm.at[src_t], rows_t, gsem[0]).wait()
    _scale_rows(rows_t, ev_t, TE // 16)
    pltpu.sync_copy(rows_t, acc.at[dst_t], add=True)

    plsc.subcore_barrier()

    pltpu.sync_copy(acc.at[pl.ds(row0, RPT)], out_hbm.at[cid, pl.ds(row0, RPT)])

    @pl.when(sid == NS - 1)
    def _wtail():
        pltpu.sync_copy(acc.at[pl.ds(NS * RPT, TAIL)],
                        out_hbm.at[cid, pl.ds(NS * RPT, TAIL)])


def kernel(x, edge_index, edge_values, W):
    xw = _matmul(x, W)
    dst = edge_index[0]
    src = edge_index[1]
    partials = _sc_aggregate(src, dst, edge_values, xw)
    return _combine(partials)
